# baseline (speedup 1.0000x reference)
import jax
import jax.numpy as jnp
from jax.experimental import pallas as pl


def _bn(x, gamma, beta):
    mean = jnp.mean(x, axis=0)
    var = jnp.var(x, axis=0)
    return gamma * (x - mean) / jnp.sqrt(var + 1e-5) + beta


def _copy_kernel(x_ref, o_ref):
    o_ref[...] = x_ref[...]


def kernel(h, e, edge_index, W_A, b_A, W_B, b_B, W_C, b_C, W_D, b_D, W_E, b_E,
           ffh_W1, ffh_b1, ffh_W2, ffh_b2, ffe_W1, ffe_b1, ffe_W2, ffe_b2,
           g1h_g, g1h_b, g1e_g, g1e_b, g2h_g, g2h_b, g2e_g, g2e_b):
    h = pl.pallas_call(_copy_kernel, out_shape=jax.ShapeDtypeStruct(h.shape, h.dtype))(h)
    src = edge_index[0]
    dst = edge_index[1]
    h_in = h
    e_in = e
    h = _bn(h, g1h_g, g1h_b)
    e = _bn(e, g1e_g, g1e_b)
    Ah = h @ W_A + b_A
    Bh = h @ W_B + b_B
    Ch = h @ W_C + b_C
    Dh = h @ W_D + b_D
    Ee = e @ W_E + b_E
    e_new = Ch[src] + Dh[dst] + Ee
    sigma = jax.nn.sigmoid(e_new)
    N = h.shape[0]
    sum_sigma_h = jax.ops.segment_sum(Bh[src] * sigma, dst, num_segments=N)
    sum_sigma = jax.ops.segment_sum(sigma, dst, num_segments=N)
    h = Ah + sum_sigma_h / (sum_sigma + 1e-10)
    e = e_new
    h = h_in + h
    e = e_in + e
    h_in2 = h
    e_in2 = e
    h = _bn(h, g2h_g, g2h_b)
    e = _bn(e, g2e_g, g2e_b)
    h = jax.nn.relu(h @ ffh_W1 + ffh_b1) @ ffh_W2 + ffh_b2
    e = jax.nn.relu(e @ ffe_W1 + ffe_b1) @ ffe_W2 + ffe_b2
    h = h_in2 + h
    e = e_in2 + e
    return (h, e)


# speedup vs baseline: 1.2906x; 1.2906x over previous
import functools

import jax
import jax.numpy as jnp
from jax import lax
from jax.experimental import pallas as pl
from jax.experimental.pallas import tpu as pltpu
from jax.experimental.pallas import tpu_sc as plsc

N_NODES = 10000
E_EDGES = 320000
D = 128
NC = 2
NS = 16
NW = NC * NS
L = 16

EPW = E_EDGES // NW
CHA = 80
NCHA = EPW // CHA

EPT = E_EDGES // NS
CHB = 80
NCHB = EPT // CHB

NP = 10240
PERT = NP // NS


def _h_pre_body(h_ref, g_ref, b_ref, wa_ref, ba_ref, wb_ref, bb_ref,
                wc_ref, bc_ref, wd_ref, bd_ref,
                ah_ref, bh_ref, ch_ref, dh_ref):
    x = h_ref[...]
    mean = jnp.mean(x, axis=0, keepdims=True)
    xc = x - mean
    var = jnp.mean(xc * xc, axis=0, keepdims=True)
    xn = g_ref[...] * xc * jax.lax.rsqrt(var + 1e-5) + b_ref[...]
    ah_ref[...] = jnp.dot(xn, wa_ref[...], preferred_element_type=jnp.float32) + ba_ref[...]
    bh_ref[...] = jnp.dot(xn, wb_ref[...], preferred_element_type=jnp.float32) + bb_ref[...]
    ch_ref[...] = jnp.dot(xn, wc_ref[...], preferred_element_type=jnp.float32) + bc_ref[...]
    dh_ref[...] = jnp.dot(xn, wd_ref[...], preferred_element_type=jnp.float32) + bd_ref[...]


def _colstats_body(x_ref, o_ref):
    i = pl.program_id(0)
    x = x_ref[...]
    s1 = jnp.sum(x, axis=0, keepdims=True)
    s2 = jnp.sum(x * x, axis=0, keepdims=True)
    blk = jnp.concatenate([s1, s2, jnp.zeros((6, D), jnp.float32)], axis=0)

    @pl.when(i == 0)
    def _init():
        o_ref[...] = blk

    @pl.when(i != 0)
    def _acc():
        o_ref[...] += blk


def _e_pre_body(x_ref, st_ref, g_ref, b_ref, we_ref, be_ref, o_ref):
    x = x_ref[...]
    mean = st_ref[0:1, :] / E_EDGES
    var = st_ref[1:2, :] / E_EDGES - mean * mean
    xn = g_ref[...] * (x - mean) * jax.lax.rsqrt(var + 1e-5) + b_ref[...]
    o_ref[...] = jnp.dot(xn, we_ref[...], preferred_element_type=jnp.float32) + be_ref[...]


def _h_post_body(ah_ref, acch_ref, accs_ref, hin_ref, g_ref, b_ref,
                 w1_ref, b1_ref, w2_ref, b2_ref, o_ref):
    hmid = ah_ref[...] + acch_ref[...] / (accs_ref[...] + 1e-10)
    h2 = hin_ref[...] + hmid
    mean = jnp.mean(h2, axis=0, keepdims=True)
    xc = h2 - mean
    var = jnp.mean(xc * xc, axis=0, keepdims=True)
    xn = g_ref[...] * xc * jax.lax.rsqrt(var + 1e-5) + b_ref[...]
    f = jnp.maximum(jnp.dot(xn, w1_ref[...], preferred_element_type=jnp.float32) + b1_ref[...], 0.0)
    o_ref[...] = h2 + jnp.dot(f, w2_ref[...], preferred_element_type=jnp.float32) + b2_ref[...]


def _e_post_body(x_ref, s1_ref, s2_ref, g_ref, b_ref, w1_ref, b1_ref, w2_ref, b2_ref, o_ref):
    x = x_ref[...]
    mean = jnp.sum(s1_ref[...], axis=0, keepdims=True) / E_EDGES
    var = jnp.sum(s2_ref[...], axis=0, keepdims=True) / E_EDGES - mean * mean
    xn = g_ref[...] * (x - mean) * jax.lax.rsqrt(var + 1e-5) + b_ref[...]
    f = jnp.maximum(jnp.dot(xn, w1_ref[...], preferred_element_type=jnp.float32) + b1_ref[...], 0.0)
    o_ref[...] = x + jnp.dot(f, w2_ref[...], preferred_element_type=jnp.float32) + b2_ref[...]


def _sc_pass_a_body(src_hbm, dst_hbm, ch_hbm, dh_hbm, bh_hbm, ee_hbm, ein_hbm,
               e2_hbm, ps_hbm, st_hbm,
               src_v, dst_v, ch_v, dh_v, bh_v, ee_v, ein_v, st_v,
               sem0, sem1, sem2, sem3, sem4):
    c = lax.axis_index("c")
    s = lax.axis_index("s")
    wid = s * NC + c
    wbase = wid * EPW

    def _zrow(i, _):
        for j in range(D // L):
            st_v[i, pl.ds(j * L, L)] = jnp.zeros((L,), jnp.float32)
        return 0
    lax.fori_loop(0, 8, _zrow, 0)

    def _chunk(k, _):
        base = pl.multiple_of(wbase + k * CHA, 8)
        pltpu.sync_copy(src_hbm.at[pl.ds(base, CHA)], src_v)
        pltpu.sync_copy(dst_hbm.at[pl.ds(base, CHA)], dst_v)
        cp0 = pltpu.async_copy(ch_hbm.at[src_v], ch_v, sem0)
        cp1 = pltpu.async_copy(dh_hbm.at[dst_v], dh_v, sem1)
        cp2 = pltpu.async_copy(bh_hbm.at[src_v], bh_v, sem2)
        cp3 = pltpu.async_copy(ee_hbm.at[pl.ds(base, CHA)], ee_v, sem3)
        cp4 = pltpu.async_copy(ein_hbm.at[pl.ds(base, CHA)], ein_v, sem4)
        cp0.wait()
        cp1.wait()
        cp2.wait()
        cp3.wait()
        cp4.wait()

        def _row(r, _):
            for j in range(D // L):
                sl = pl.ds(j * L, L)
                t = ch_v[r, sl] + dh_v[r, sl] + ee_v[r, sl]
                sg = 1.0 / (1.0 + jnp.exp(-t))
                e2 = t + ein_v[r, sl]
                ee_v[r, sl] = e2
                ch_v[r, sl] = bh_v[r, sl] * sg
                dh_v[r, sl] = sg
                plsc.addupdate(st_v.at[0, sl], e2)
                plsc.addupdate(st_v.at[1, sl], e2 * e2)
            return 0
        lax.fori_loop(0, CHA, _row, 0)

        pltpu.sync_copy(ee_v, e2_hbm.at[pl.ds(base, CHA)])
        pltpu.sync_copy(ch_v, ps_hbm.at[0, pl.ds(base, CHA)])
        pltpu.sync_copy(dh_v, ps_hbm.at[1, pl.ds(base, CHA)])
        return 0

    lax.fori_loop(0, NCHA, _chunk, 0)
    pltpu.sync_copy(st_v, st_hbm.at[wid])


def _sc_pass_b_body(dst_hbm, ps_hbm, out_hbm, idx_v, rows_v, zero_v, acc_sh):
    c = lax.axis_index("c")
    s = lax.axis_index("s")

    def _zrow(i, _):
        for j in range(D // L):
            zero_v[i, pl.ds(j * L, L)] = jnp.zeros((L,), jnp.float32)
        return 0
    lax.fori_loop(0, 64, _zrow, 0)

    def _zcopy(k, _):
        pltpu.sync_copy(zero_v, acc_sh.at[pl.ds(s * PERT + k * 64, 64)])
        return 0
    lax.fori_loop(0, PERT // 64, _zcopy, 0)
    plsc.subcore_barrier()

    tbase = s * EPT

    def _chunk(k, _):
        base = pl.multiple_of(tbase + k * CHB, 8)
        pltpu.sync_copy(dst_hbm.at[pl.ds(base, CHB)], idx_v)
        pltpu.sync_copy(ps_hbm.at[c, pl.ds(base, CHB)], rows_v)
        pltpu.sync_copy(rows_v, acc_sh.at[idx_v], add=True)
        return 0

    lax.fori_loop(0, NCHB, _chunk, 0)
    plsc.subcore_barrier()
    pltpu.sync_copy(acc_sh.at[pl.ds(s * PERT, PERT)],
                    out_hbm.at[c, pl.ds(s * PERT, PERT)])


@functools.lru_cache(maxsize=None)
def _sc_kernels():
    mesh = plsc.VectorSubcoreMesh(core_axis_name="c", subcore_axis_name="s",
                                  num_cores=NC, num_subcores=NS)
    pass_a = pl.kernel(
        _sc_pass_a_body,
        out_type=(
            jax.ShapeDtypeStruct((E_EDGES, D), jnp.float32),
            jax.ShapeDtypeStruct((2, E_EDGES, D), jnp.float32),
            jax.ShapeDtypeStruct((NW, 8, D), jnp.float32),
        ),
        mesh=mesh,
        scratch_types=[
            pltpu.VMEM((CHA,), jnp.int32),
            pltpu.VMEM((CHA,), jnp.int32),
            pltpu.VMEM((CHA, D), jnp.float32),
            pltpu.VMEM((CHA, D), jnp.float32),
            pltpu.VMEM((CHA, D), jnp.float32),
            pltpu.VMEM((CHA, D), jnp.float32),
            pltpu.VMEM((CHA, D), jnp.float32),
            pltpu.VMEM((8, D), jnp.float32),
            pltpu.SemaphoreType.DMA,
            pltpu.SemaphoreType.DMA,
            pltpu.SemaphoreType.DMA,
            pltpu.SemaphoreType.DMA,
            pltpu.SemaphoreType.DMA,
        ],
    )
    pass_b = pl.kernel(
        _sc_pass_b_body,
        out_type=jax.ShapeDtypeStruct((2, NP, D), jnp.float32),
        mesh=mesh,
        scratch_types=[
            pltpu.VMEM((CHB,), jnp.int32),
            pltpu.VMEM((CHB, D), jnp.float32),
            pltpu.VMEM((64, D), jnp.float32),
            pltpu.VMEM_SHARED((NP, D), jnp.float32),
        ],
    )
    return pass_a, pass_b


def kernel(h, e, edge_index, W_A, b_A, W_B, b_B, W_C, b_C, W_D, b_D, W_E, b_E,
           ffh_W1, ffh_b1, ffh_W2, ffh_b2, ffe_W1, ffe_b1, ffe_W2, ffe_b2,
           g1h_g, g1h_b, g1e_g, g1e_b, g2h_g, g2h_b, g2e_g, g2e_b):
    src = edge_index[0]
    dst = edge_index[1]
    row = lambda v: v.reshape(1, D)

    ah, bh, ch, dh = pl.pallas_call(
        _h_pre_body,
        out_shape=[jax.ShapeDtypeStruct((N_NODES, D), jnp.float32)] * 4,
    )(h, row(g1h_g), row(g1h_b), W_A, row(b_A), W_B, row(b_B),
      W_C, row(b_C), W_D, row(b_D))

    BLK = 2000
    grid = E_EDGES // BLK
    estats = pl.pallas_call(
        _colstats_body,
        grid=(grid,),
        in_specs=[pl.BlockSpec((BLK, D), lambda i: (i, 0))],
        out_specs=pl.BlockSpec((8, D), lambda i: (0, 0)),
        out_shape=jax.ShapeDtypeStruct((8, D), jnp.float32),
    )(e)

    ee = pl.pallas_call(
        _e_pre_body,
        grid=(grid,),
        in_specs=[
            pl.BlockSpec((BLK, D), lambda i: (i, 0)),
            pl.BlockSpec((8, D), lambda i: (0, 0)),
            pl.BlockSpec((1, D), lambda i: (0, 0)),
            pl.BlockSpec((1, D), lambda i: (0, 0)),
            pl.BlockSpec((D, D), lambda i: (0, 0)),
            pl.BlockSpec((1, D), lambda i: (0, 0)),
        ],
        out_specs=pl.BlockSpec((BLK, D), lambda i: (i, 0)),
        out_shape=jax.ShapeDtypeStruct((E_EDGES, D), jnp.float32),
    )(e, estats, row(g1e_g), row(g1e_b), W_E, row(b_E))

    sc_pass_a, sc_pass_b = _sc_kernels()
    e2, ps, e2stats = sc_pass_a(src, dst, ch, dh, bh, ee, e)
    accs = sc_pass_b(dst, ps)

    h_out = pl.pallas_call(
        _h_post_body,
        out_shape=jax.ShapeDtypeStruct((N_NODES, D), jnp.float32),
    )(ah, accs[0, :N_NODES], accs[1, :N_NODES], h, row(g2h_g), row(g2h_b),
      ffh_W1, row(ffh_b1), ffh_W2, row(ffh_b2))

    e_out = pl.pallas_call(
        _e_post_body,
        grid=(grid,),
        in_specs=[
            pl.BlockSpec((BLK, D), lambda i: (i, 0)),
            pl.BlockSpec((NW, D), lambda i: (0, 0)),
            pl.BlockSpec((NW, D), lambda i: (0, 0)),
            pl.BlockSpec((1, D), lambda i: (0, 0)),
            pl.BlockSpec((1, D), lambda i: (0, 0)),
            pl.BlockSpec((D, D), lambda i: (0, 0)),
            pl.BlockSpec((1, D), lambda i: (0, 0)),
            pl.BlockSpec((D, D), lambda i: (0, 0)),
            pl.BlockSpec((1, D), lambda i: (0, 0)),
        ],
        out_specs=pl.BlockSpec((BLK, D), lambda i: (i, 0)),
        out_shape=jax.ShapeDtypeStruct((E_EDGES, D), jnp.float32),
    )(e2, e2stats[:, 0, :], e2stats[:, 1, :], row(g2e_g), row(g2e_b),
      ffe_W1, row(ffe_b1), ffe_W2, row(ffe_b2))

    return (h_out, e_out)
